# Initial kernel scaffold; baseline (speedup 1.0000x reference)
import jax
import jax.numpy as jnp
from jax.experimental import pallas as pl


def kernel(z, src_mask, dst_mask, neg_mask, W_src, b_src, W_dst, b_dst, W_out, b_out):
    raise NotImplementedError("write your pallas kernel here")


# speedup vs baseline: 1.7662x; 1.7662x over previous
import functools

import jax
import jax.numpy as jnp
from jax import lax
from jax.experimental import pallas as pl
from jax.experimental.pallas import tpu as pltpu
from jax.experimental.pallas import tpu_sc as plsc

DIM = 256
LANES = 16
K16 = DIM // LANES
NC, NS = 2, 16
NW = NC * NS
CHUNK = 128


def _tc_body(z_ref, ws_ref, wd_ref, bias_ref, a_ref, b_ref):
    zb = z_ref[...]
    a_ref[...] = (
        jnp.dot(zb, ws_ref[...], preferred_element_type=jnp.float32)
        + bias_ref[...]
    )
    b_ref[...] = jnp.dot(zb, wd_ref[...], preferred_element_type=jnp.float32)


def _node_transform(z, W_src, W_dst, bias2d):
    n, d = z.shape
    blk = 1000
    assert n % blk == 0 and blk % 8 == 0
    grid = (n // blk,)
    return pl.pallas_call(
        _tc_body,
        grid=grid,
        in_specs=[
            pl.BlockSpec((blk, d), lambda i: (i, 0)),
            pl.BlockSpec((d, d), lambda i: (0, 0)),
            pl.BlockSpec((d, d), lambda i: (0, 0)),
            pl.BlockSpec((1, d), lambda i: (0, 0)),
        ],
        out_specs=[
            pl.BlockSpec((blk, d), lambda i: (i, 0)),
            pl.BlockSpec((blk, d), lambda i: (i, 0)),
        ],
        out_shape=[
            jax.ShapeDtypeStruct((n, d), jnp.float32),
            jax.ShapeDtypeStruct((n, d), jnp.float32),
        ],
    )(z, W_src, W_dst, bias2d)


@functools.lru_cache(maxsize=None)
def _make_edge_kernel(epad):
    chunks_per_w = epad // (NW * CHUNK)
    mesh = plsc.VectorSubcoreMesh(core_axis_name="c", subcore_axis_name="s")

    @functools.partial(
        pl.kernel,
        out_type=[
            jax.ShapeDtypeStruct((epad,), jnp.float32),
            jax.ShapeDtypeStruct((epad,), jnp.float32),
        ],
        mesh=mesh,
        compiler_params=pltpu.CompilerParams(needs_layout_passes=False),
        scratch_types=[
            pltpu.VMEM((CHUNK,), jnp.int32),
            pltpu.VMEM((CHUNK,), jnp.int32),
            pltpu.VMEM((CHUNK,), jnp.int32),
            pltpu.VMEM((CHUNK, DIM), jnp.float32),
            pltpu.VMEM((CHUNK, DIM), jnp.float32),
            pltpu.VMEM((CHUNK, DIM), jnp.float32),
            pltpu.VMEM((CHUNK,), jnp.float32),
            pltpu.VMEM((CHUNK,), jnp.float32),
            pltpu.VMEM((DIM + LANES,), jnp.float32),
            pltpu.SemaphoreType.DMA,
            pltpu.SemaphoreType.DMA,
            pltpu.SemaphoreType.DMA,
        ],
    )
    def edge_kernel(a_hbm, b_hbm, src_hbm, dst_hbm, neg_hbm, wtab_hbm,
                    pos_hbm, negout_hbm,
                    sidx, didx, nidx, a_rows, b_rows, c_rows,
                    outp, outn, wtab, sem0, sem1, sem2):
        wid = lax.axis_index("s") * NC + lax.axis_index("c")
        pltpu.sync_copy(wtab_hbm, wtab)

        def chunk_body(i, carry):
            base = (wid * chunks_per_w + i) * CHUNK
            pltpu.sync_copy(src_hbm.at[pl.ds(base, CHUNK)], sidx)
            pltpu.sync_copy(dst_hbm.at[pl.ds(base, CHUNK)], didx)
            pltpu.sync_copy(neg_hbm.at[pl.ds(base, CHUNK)], nidx)
            cp0 = pltpu.async_copy(a_hbm.at[sidx], a_rows, sem0)
            cp1 = pltpu.async_copy(b_hbm.at[didx], b_rows, sem1)
            cp2 = pltpu.async_copy(b_hbm.at[nidx], c_rows, sem2)
            cp0.wait()
            cp1.wait()
            cp2.wait()

            bvec = wtab[pl.ds(DIM, LANES)]
            lane = lax.iota(jnp.int32, LANES)

            def group_body(g, c):
                vecp = jnp.zeros((LANES,), jnp.float32)
                vecn = jnp.zeros((LANES,), jnp.float32)
                for ee in range(LANES):
                    e = g * LANES + ee
                    accp = bvec
                    accn = bvec
                    for k in range(K16):
                        a = a_rows[e, pl.ds(k * LANES, LANES)]
                        bb = b_rows[e, pl.ds(k * LANES, LANES)]
                        cc = c_rows[e, pl.ds(k * LANES, LANES)]
                        w = wtab[pl.ds(k * LANES, LANES)]
                        accp = accp + jnp.maximum(a + bb, 0.0) * w
                        accn = accn + jnp.maximum(a + cc, 0.0) * w
                    sp = lax.reduce_sum(accp, axes=(0,))
                    sn = lax.reduce_sum(accn, axes=(0,))
                    vecp = jnp.where(lane == ee, sp, vecp)
                    vecn = jnp.where(lane == ee, sn, vecn)
                outp[pl.ds(g * LANES, LANES)] = vecp
                outn[pl.ds(g * LANES, LANES)] = vecn
                return c

            lax.fori_loop(0, CHUNK // LANES, group_body, 0)

            pltpu.sync_copy(outp, pos_hbm.at[pl.ds(base, CHUNK)])
            pltpu.sync_copy(outn, negout_hbm.at[pl.ds(base, CHUNK)])
            return carry

        lax.fori_loop(0, chunks_per_w, chunk_body, 0)

    return edge_kernel


def kernel(z, src_mask, dst_mask, neg_mask, W_src, b_src, W_dst, b_dst,
           W_out, b_out):
    n, d = z.shape
    e = src_mask.shape[0]

    bias2d = (b_src + b_dst).reshape(1, d)
    A, B = _node_transform(z, W_src, W_dst, bias2d)

    wtab = jnp.concatenate(
        [W_out.reshape(-1),
         jnp.pad(b_out.reshape(-1)[:1], (0, LANES - 1))]
    ).astype(jnp.float32)

    stride = NW * CHUNK
    epad = ((e + stride - 1) // stride) * stride
    pad = epad - e
    src_p = jnp.concatenate([src_mask.astype(jnp.int32), jnp.zeros((pad,), jnp.int32)])
    dst_p = jnp.concatenate([dst_mask.astype(jnp.int32), jnp.zeros((pad,), jnp.int32)])
    neg_p = jnp.concatenate([neg_mask.astype(jnp.int32), jnp.zeros((pad,), jnp.int32)])

    pos_flat, neg_flat = _make_edge_kernel(epad)(
        A, B, src_p, dst_p, neg_p, wtab)

    return (pos_flat[:e].reshape(e, 1), neg_flat[:e].reshape(e, 1))


# speedup vs baseline: 2.3116x; 1.3088x over previous
import functools

import jax
import jax.numpy as jnp
from jax import lax
from jax.experimental import pallas as pl
from jax.experimental.pallas import tpu as pltpu
from jax.experimental.pallas import tpu_sc as plsc

DIM = 256
LANES = 16
K16 = DIM // LANES
NC, NS = 2, 16
NW = NC * NS
CHUNK = 64


def _tc_body(z_ref, ws_ref, wd_ref, bias_ref, a_ref, b_ref):
    zb = z_ref[...]
    a_ref[...] = (
        jnp.dot(zb, ws_ref[...], preferred_element_type=jnp.float32)
        + bias_ref[...]
    )
    b_ref[...] = jnp.dot(zb, wd_ref[...], preferred_element_type=jnp.float32)


def _node_transform(z, W_src, W_dst, bias2d):
    n, d = z.shape
    blk = 1000
    assert n % blk == 0 and blk % 8 == 0
    grid = (n // blk,)
    return pl.pallas_call(
        _tc_body,
        grid=grid,
        in_specs=[
            pl.BlockSpec((blk, d), lambda i: (i, 0)),
            pl.BlockSpec((d, d), lambda i: (0, 0)),
            pl.BlockSpec((d, d), lambda i: (0, 0)),
            pl.BlockSpec((1, d), lambda i: (0, 0)),
        ],
        out_specs=[
            pl.BlockSpec((blk, d), lambda i: (i, 0)),
            pl.BlockSpec((blk, d), lambda i: (i, 0)),
        ],
        out_shape=[
            jax.ShapeDtypeStruct((n, d), jnp.float32),
            jax.ShapeDtypeStruct((n, d), jnp.float32),
        ],
    )(z, W_src, W_dst, bias2d)


@functools.lru_cache(maxsize=None)
def _make_edge_kernel(epad):
    chunks_per_w = epad // (NW * CHUNK)
    edges_per_w = chunks_per_w * CHUNK
    assert chunks_per_w % 2 == 0
    mesh = plsc.VectorSubcoreMesh(core_axis_name="c", subcore_axis_name="s")

    @functools.partial(
        pl.kernel,
        out_type=[
            jax.ShapeDtypeStruct((epad,), jnp.float32),
            jax.ShapeDtypeStruct((epad,), jnp.float32),
        ],
        mesh=mesh,
        compiler_params=pltpu.CompilerParams(needs_layout_passes=False),
        scratch_types=[
            pltpu.VMEM((edges_per_w,), jnp.int32),
            pltpu.VMEM((edges_per_w,), jnp.int32),
            pltpu.VMEM((edges_per_w,), jnp.int32),
            pltpu.VMEM((2, CHUNK, DIM), jnp.float32),
            pltpu.VMEM((2, CHUNK, DIM), jnp.float32),
            pltpu.VMEM((2, CHUNK, DIM), jnp.float32),
            pltpu.VMEM((CHUNK,), jnp.float32),
            pltpu.VMEM((CHUNK,), jnp.float32),
            pltpu.VMEM((DIM + LANES,), jnp.float32),
            pltpu.SemaphoreType.DMA,
            pltpu.SemaphoreType.DMA,
        ],
    )
    def edge_kernel(a_hbm, b_hbm, src_hbm, dst_hbm, neg_hbm, wtab_hbm,
                    pos_hbm, negout_hbm,
                    sidx, didx, nidx, a_rows, b_rows, c_rows,
                    outp, outn, wtab, sem0, sem1):
        wid = lax.axis_index("s") * NC + lax.axis_index("c")
        ebase = wid * edges_per_w
        pltpu.sync_copy(wtab_hbm, wtab)
        pltpu.sync_copy(src_hbm.at[pl.ds(ebase, edges_per_w)], sidx)
        pltpu.sync_copy(dst_hbm.at[pl.ds(ebase, edges_per_w)], didx)
        pltpu.sync_copy(neg_hbm.at[pl.ds(ebase, edges_per_w)], nidx)

        def fire(par, ci, sem):
            off = ci * CHUNK
            pltpu.async_copy(
                a_hbm.at[sidx.at[pl.ds(off, CHUNK)]], a_rows.at[par], sem)
            pltpu.async_copy(
                b_hbm.at[didx.at[pl.ds(off, CHUNK)]], b_rows.at[par], sem)
            pltpu.async_copy(
                b_hbm.at[nidx.at[pl.ds(off, CHUNK)]], c_rows.at[par], sem)

        def drain(par, ci, sem):
            off = ci * CHUNK
            pltpu.make_async_copy(
                a_hbm.at[sidx.at[pl.ds(off, CHUNK)]], a_rows.at[par],
                sem).wait()
            pltpu.make_async_copy(
                b_hbm.at[didx.at[pl.ds(off, CHUNK)]], b_rows.at[par],
                sem).wait()
            pltpu.make_async_copy(
                b_hbm.at[nidx.at[pl.ds(off, CHUNK)]], c_rows.at[par],
                sem).wait()

        bvec = wtab[pl.ds(DIM, LANES)]
        lane = lax.iota(jnp.int32, LANES)

        def compute(par, ci):
            def group_body(g, c):
                vecp = jnp.zeros((LANES,), jnp.float32)
                vecn = jnp.zeros((LANES,), jnp.float32)
                for ee in range(LANES):
                    e = g * LANES + ee
                    accp = bvec
                    accn = bvec
                    for k in range(K16):
                        a = a_rows[par, e, pl.ds(k * LANES, LANES)]
                        bb = b_rows[par, e, pl.ds(k * LANES, LANES)]
                        cc = c_rows[par, e, pl.ds(k * LANES, LANES)]
                        w = wtab[pl.ds(k * LANES, LANES)]
                        accp = accp + jnp.maximum(a + bb, 0.0) * w
                        accn = accn + jnp.maximum(a + cc, 0.0) * w
                    sp = lax.reduce_sum(accp, axes=(0,))
                    sn = lax.reduce_sum(accn, axes=(0,))
                    vecp = jnp.where(lane == ee, sp, vecp)
                    vecn = jnp.where(lane == ee, sn, vecn)
                outp[pl.ds(g * LANES, LANES)] = vecp
                outn[pl.ds(g * LANES, LANES)] = vecn
                return c

            lax.fori_loop(0, CHUNK // LANES, group_body, 0)
            base = ebase + ci * CHUNK
            pltpu.sync_copy(outp, pos_hbm.at[pl.ds(base, CHUNK)])
            pltpu.sync_copy(outn, negout_hbm.at[pl.ds(base, CHUNK)])

        fire(0, 0, sem0)

        def pair_body(i2, c):
            ca = 2 * i2
            fire(1, ca + 1, sem1)
            drain(0, ca, sem0)
            compute(0, ca)

            @pl.when(ca + 2 < chunks_per_w)
            def _():
                fire(0, ca + 2, sem0)

            drain(1, ca + 1, sem1)
            compute(1, ca + 1)
            return c

        lax.fori_loop(0, chunks_per_w // 2, pair_body, 0)

    return edge_kernel


def kernel(z, src_mask, dst_mask, neg_mask, W_src, b_src, W_dst, b_dst,
           W_out, b_out):
    n, d = z.shape
    e = src_mask.shape[0]

    bias2d = (b_src + b_dst).reshape(1, d)
    A, B = _node_transform(z, W_src, W_dst, bias2d)

    wtab = jnp.concatenate(
        [W_out.reshape(-1),
         jnp.pad(b_out.reshape(-1)[:1], (0, LANES - 1))]
    ).astype(jnp.float32)

    stride = NW * CHUNK * 2
    epad = ((e + stride - 1) // stride) * stride
    pad = epad - e
    src_p = jnp.concatenate([src_mask.astype(jnp.int32), jnp.zeros((pad,), jnp.int32)])
    dst_p = jnp.concatenate([dst_mask.astype(jnp.int32), jnp.zeros((pad,), jnp.int32)])
    neg_p = jnp.concatenate([neg_mask.astype(jnp.int32), jnp.zeros((pad,), jnp.int32)])

    pos_flat, neg_flat = _make_edge_kernel(epad)(
        A, B, src_p, dst_p, neg_p, wtab)

    return (pos_flat[:e].reshape(e, 1), neg_flat[:e].reshape(e, 1))
